# baseline (speedup 1.0000x reference)
import jax
import jax.numpy as jnp
from jax import lax
from jax.experimental import pallas as pl
from jax.experimental.pallas import tpu as pltpu
from jax.experimental.pallas import tpu_sc as plsc

N_NODES = 10000
N_EDGES = 320000
D = 128

NC = 2
NS = 16
NW = NC * NS

NP = 10240
CR = NP // D
CHUNK = 72
NCH0 = 174
NCH1 = 114
EPW0 = NCH0 * CHUNK
EPW1 = NCH1 * CHUNK
EPAD = NS * (EPW0 + EPW1)
TOTCH = EPAD // CHUNK
EPW_DEG = EPAD // NW
RPT = NP // NS
ZR = 32
DCH = 2592

_SC_PARAMS = pltpu.CompilerParams(needs_layout_passes=False)
_MESH = plsc.VectorSubcoreMesh(core_axis_name="c", subcore_axis_name="s")


def _make_degree():
  def body(dst, cnt_out, dst_v, cnt_v, rowid_v, cacc):
    cid = lax.axis_index("c")
    sid = lax.axis_index("s")
    wid = cid * NS + sid

    zero16 = jnp.zeros((16,), jnp.float32)
    ones16 = jnp.ones((16,), jnp.float32)
    for i in range(CR):
      for k in range(D // 16):
        cnt_v[i, k * 16:(k + 1) * 16] = zero16
    for k in range(CR // 16):
      rowid_v[k * 16:(k + 1) * 16] = (
          lax.iota(jnp.int32, 16) + jnp.int32(k * 16))
    @pl.when(sid == 0)
    def _zero_cacc():
      pltpu.sync_copy(cnt_v, cacc)
    plsc.subcore_barrier()

    estart = wid * EPW_DEG

    def load_body(i, carry):
      pltpu.sync_copy(dst.at[pl.ds(estart + i * DCH, DCH)], dst_v)
      for k in range(DCH // 16):
        dvec = dst_v[k * 16:(k + 1) * 16]
        plsc.addupdate_scatter(cnt_v, [dvec >> 7, dvec & 127], ones16)
      return carry

    lax.fori_loop(0, EPW_DEG // DCH, load_body, 0)
    pltpu.sync_copy(cnt_v, cacc.at[rowid_v], add=True)
    plsc.subcore_barrier()

    @pl.when(sid == 0)
    def _copy_out():
      for t in range(CR // 16):
        pltpu.sync_copy(cacc.at[pl.ds(t * 16, 16)], cnt_v.at[pl.ds(0, 16)])
        pltpu.sync_copy(cnt_v.at[pl.ds(0, 16)],
                        cnt_out.at[pl.ds(cid * CR + t * 16, 16)])

  return pl.kernel(
      body,
      out_type=jax.ShapeDtypeStruct((NC * CR, D), jnp.float32),
      mesh=_MESH,
      scratch_types=[
          pltpu.VMEM((DCH,), jnp.int32),
          pltpu.VMEM((CR, D), jnp.float32),
          pltpu.VMEM((CR,), jnp.int32),
          pltpu.VMEM_SHARED((CR, D), jnp.float32),
      ],
      compiler_params=_SC_PARAMS,
  )


def _make_agg():
  def body(feat, il, sums_out,
           ib0, ib1, ib2, r0, r1, r2, acc, gs0, gs1, gs2, ss0, ss1, ss2):
    cid = lax.axis_index("c")
    sid = lax.axis_index("s")
    si = (ib0.at[0], ib1.at[0], ib2.at[0])
    di = (ib0.at[1], ib1.at[1], ib2.at[1])
    ib = (ib0, ib1, ib2)
    rows = (r0, r1, r2)
    gsem = (gs0, gs1, gs2)
    ssem = (ss0, ss1, ss2)

    zero16 = jnp.zeros((16,), jnp.float32)
    for i in range(ZR):
      for k in range(D // 16):
        r0[i, k * 16:(k + 1) * 16] = zero16
    rbase = sid * RPT
    for t in range(RPT // ZR):
      pltpu.sync_copy(r0.at[pl.ds(0, ZR)], acc.at[pl.ds(rbase + t * ZR, ZR)])
    plsc.subcore_barrier()

    nch = jnp.where(cid == 0, NCH0, NCH1)
    cstart = jnp.where(cid == 0, sid * NCH0, NS * NCH0 + sid * NCH1)

    def load_idx(j, p):
      pltpu.sync_copy(il.at[cstart + j], ib[p])

    load_idx(0, 0)
    pltpu.async_copy(feat.at[si[0]], rows[0], gsem[0])
    load_idx(1, 1)
    pltpu.async_copy(feat.at[si[1]], rows[1], gsem[1])

    def outer(g, carry):
      for b in range(3):
        j = 3 * g + b
        bn = (b + 2) % 3
        pltpu.make_async_copy(feat.at[si[b]], rows[b], gsem[b]).wait()
        pltpu.async_copy(rows[b], acc.at[di[b]], ssem[b], add=True)

        def _prefetch(j=j, b=b, bn=bn):
          def _wait_prev():
            pltpu.make_async_copy(rows[bn], acc.at[di[bn]], ssem[bn]).wait()
          if b == 0:
            pl.when(j >= 1)(_wait_prev)
          else:
            _wait_prev()
          load_idx(j + 2, bn)
          pltpu.async_copy(feat.at[si[bn]], rows[bn], gsem[bn])

        pl.when(j + 2 < nch)(_prefetch)
      return carry

    lax.fori_loop(0, nch // 3, outer, 0)
    pltpu.make_async_copy(rows[0], acc.at[di[0]], ssem[0]).wait()
    pltpu.make_async_copy(rows[1], acc.at[di[1]], ssem[1]).wait()
    pltpu.make_async_copy(rows[2], acc.at[di[2]], ssem[2]).wait()
    plsc.subcore_barrier()

    obase = cid * NP + rbase
    for t in range(RPT // CHUNK):
      pltpu.sync_copy(acc.at[pl.ds(rbase + t * CHUNK, CHUNK)], r0)
      pltpu.sync_copy(r0, sums_out.at[pl.ds(obase + t * CHUNK, CHUNK)])
    tail = RPT - (RPT // CHUNK) * CHUNK
    toff = (RPT // CHUNK) * CHUNK
    if tail:
      pltpu.sync_copy(acc.at[pl.ds(rbase + toff, tail)], r0.at[pl.ds(0, tail)])
      pltpu.sync_copy(r0.at[pl.ds(0, tail)],
                      sums_out.at[pl.ds(obase + toff, tail)])

  return pl.kernel(
      body,
      out_type=jax.ShapeDtypeStruct((NC * NP, D), jnp.float32),
      mesh=_MESH,
      scratch_types=[
          pltpu.VMEM((2, CHUNK), jnp.int32),
          pltpu.VMEM((2, CHUNK), jnp.int32),
          pltpu.VMEM((2, CHUNK), jnp.int32),
          pltpu.VMEM((CHUNK, D), jnp.float32),
          pltpu.VMEM((CHUNK, D), jnp.float32),
          pltpu.VMEM((CHUNK, D), jnp.float32),
          pltpu.VMEM_SHARED((NP, D), jnp.float32),
          pltpu.SemaphoreType.DMA,
          pltpu.SemaphoreType.DMA,
          pltpu.SemaphoreType.DMA,
          pltpu.SemaphoreType.DMA,
          pltpu.SemaphoreType.DMA,
          pltpu.SemaphoreType.DMA,
      ],
      compiler_params=_SC_PARAMS,
  )


RB = 1024


def _make_combine(relu: bool):
  def body(s_ref, c_ref, x_ref, wl_ref, b_ref, wr_ref, o_ref):
    s = s_ref[0] + s_ref[1]
    c = c_ref[0] + c_ref[1]
    mean = s / jnp.maximum(c, 1.0)
    acc = lax.dot_general(mean, wl_ref[...], (((1,), (1,)), ((), ())),
                          preferred_element_type=jnp.float32)
    acc = acc + lax.dot_general(x_ref[...], wr_ref[...],
                                (((1,), (1,)), ((), ())),
                                preferred_element_type=jnp.float32)
    acc = acc + b_ref[...]
    o_ref[...] = jnp.maximum(acc, 0.0) if relu else acc

  return pl.pallas_call(
      body,
      grid=(NP // RB,),
      in_specs=[
          pl.BlockSpec((NC, RB, D), lambda i: (0, i, 0)),
          pl.BlockSpec((NC, RB, 1), lambda i: (0, i, 0)),
          pl.BlockSpec((RB, D), lambda i: (i, 0)),
          pl.BlockSpec((D, D), lambda i: (0, 0)),
          pl.BlockSpec((1, D), lambda i: (0, 0)),
          pl.BlockSpec((D, D), lambda i: (0, 0)),
      ],
      out_specs=pl.BlockSpec((RB, D), lambda i: (i, 0)),
      out_shape=jax.ShapeDtypeStruct((NP, D), jnp.float32),
  )


_degree = _make_degree()
_agg = _make_agg()
_combine_relu = _make_combine(True)
_combine_lin = _make_combine(False)


def kernel(x, edge_index, W_l1, b_l1, W_r1, W_l2, b_l2, W_r2):
  src = edge_index[0].astype(jnp.int32)
  dst = edge_index[1].astype(jnp.int32)
  pad = EPAD - N_EDGES
  src_p = jnp.concatenate([src, jnp.zeros((pad,), jnp.int32)])
  dst_p = jnp.concatenate([dst, jnp.full((pad,), NP - 8, jnp.int32)])
  x_p = jnp.pad(x, ((0, NP - N_NODES), (0, 0)))

  cnt = _degree(dst_p)
  c1 = cnt.reshape(NC, NP, 1)

  il = jnp.stack([src_p.reshape(TOTCH, CHUNK), dst_p.reshape(TOTCH, CHUNK)],
                 axis=1)
  sums1 = _agg(x_p, il)
  h = _combine_relu(sums1.reshape(NC, NP, D), c1, x_p,
                    W_l1, b_l1.reshape(1, D), W_r1)

  sums2 = _agg(h, il)
  out = _combine_lin(sums2.reshape(NC, NP, D), c1, h,
                     W_l2, b_l2.reshape(1, D), W_r2)
  return out[:N_NODES]


# speedup vs baseline: 3.1001x; 3.1001x over previous
import jax
import jax.numpy as jnp
from jax import lax
from jax.experimental import pallas as pl
from jax.experimental.pallas import tpu as pltpu
from jax.experimental.pallas import tpu_sc as plsc

N_NODES = 10000
N_EDGES = 320000
D = 128

NC = 2
NS = 16
NW = NC * NS

NP = 10240
CR = NP // D
CHUNK = 112
NCH0 = 128
NCH1 = 52
EPW0 = NCH0 * CHUNK
EPW1 = NCH1 * CHUNK
CORE1_BASE = NS * EPW0
EPAD = NS * (EPW0 + EPW1)
TOTCH = EPAD // CHUNK
EPW_DEG = EPAD // NW
RPT = NP // NS
ZR = 32
DCH = 2016

_SC_PARAMS = pltpu.CompilerParams(needs_layout_passes=False)
_MESH = plsc.VectorSubcoreMesh(core_axis_name="c", subcore_axis_name="s")


def _make_degree():
  def body(dst, cnt_out, dst_v, cnt_v, rowid_v, cacc):
    cid = lax.axis_index("c")
    sid = lax.axis_index("s")
    wid = cid * NS + sid

    zero16 = jnp.zeros((16,), jnp.float32)
    ones16 = jnp.ones((16,), jnp.float32)
    for i in range(CR):
      for k in range(D // 16):
        cnt_v[i, k * 16:(k + 1) * 16] = zero16
    for k in range(CR // 16):
      rowid_v[k * 16:(k + 1) * 16] = (
          lax.iota(jnp.int32, 16) + jnp.int32(k * 16))
    @pl.when(sid == 0)
    def _zero_cacc():
      pltpu.sync_copy(cnt_v, cacc)
    plsc.subcore_barrier()

    estart = wid * EPW_DEG

    def load_body(i, carry):
      pltpu.sync_copy(dst.at[pl.ds(estart + i * DCH, DCH)], dst_v)
      for k in range(DCH // 16):
        dvec = dst_v[k * 16:(k + 1) * 16]
        plsc.addupdate_scatter(cnt_v, [dvec >> 7, dvec & 127], ones16)
      return carry

    lax.fori_loop(0, EPW_DEG // DCH, load_body, 0)
    pltpu.sync_copy(cnt_v, cacc.at[rowid_v], add=True)
    plsc.subcore_barrier()

    @pl.when(sid == 0)
    def _copy_out():
      for t in range(CR // 16):
        pltpu.sync_copy(cacc.at[pl.ds(t * 16, 16)], cnt_v.at[pl.ds(0, 16)])
        pltpu.sync_copy(cnt_v.at[pl.ds(0, 16)],
                        cnt_out.at[pl.ds(cid * CR + t * 16, 16)])

  return pl.kernel(
      body,
      out_type=jax.ShapeDtypeStruct((NC * CR, D), jnp.float32),
      mesh=_MESH,
      scratch_types=[
          pltpu.VMEM((DCH,), jnp.int32),
          pltpu.VMEM((CR, D), jnp.float32),
          pltpu.VMEM((CR,), jnp.int32),
          pltpu.VMEM_SHARED((CR, D), jnp.float32),
      ],
      compiler_params=_SC_PARAMS,
  )


def _make_agg():
  def body(feat, il, sums_out,
           ib0, ib1, r0, r1, acc, gs0, gs1, ss0, ss1):
    cid = lax.axis_index("c")
    sid = lax.axis_index("s")
    si = (ib0.at[0], ib1.at[0])
    di = (ib0.at[1], ib1.at[1])
    ib = (ib0, ib1)
    rows = (r0, r1)
    gsem = (gs0, gs1)
    ssem = (ss0, ss1)

    zero16 = jnp.zeros((16,), jnp.float32)
    for i in range(ZR):
      for k in range(D // 16):
        r0[i, k * 16:(k + 1) * 16] = zero16
    rbase = sid * RPT
    for t in range(RPT // ZR):
      pltpu.sync_copy(r0.at[pl.ds(0, ZR)], acc.at[pl.ds(rbase + t * ZR, ZR)])
    plsc.subcore_barrier()

    nch = jnp.where(cid == 0, NCH0, NCH1)
    cstart = jnp.where(cid == 0, sid * NCH0, NS * NCH0 + sid * NCH1)

    def load_idx(j, p):
      pltpu.sync_copy(il.at[cstart + j], ib[p])

    load_idx(0, 0)
    pltpu.async_copy(feat.at[si[0]], rows[0], gsem[0])

    def outer(g, carry):
      j0 = 2 * g
      @pl.when(g >= 1)
      def _wait_prev_scatter():
        pltpu.make_async_copy(rows[1], acc.at[di[1]], ssem[1]).wait()
      load_idx(j0 + 1, 1)
      pltpu.async_copy(feat.at[si[1]], rows[1], gsem[1])
      pltpu.make_async_copy(feat.at[si[0]], rows[0], gsem[0]).wait()
      pltpu.async_copy(rows[0], acc.at[di[0]], ssem[0], add=True)
      @pl.when(j0 + 2 < nch)
      def _next_gather():
        pltpu.make_async_copy(rows[0], acc.at[di[0]], ssem[0]).wait()
        load_idx(j0 + 2, 0)
        pltpu.async_copy(feat.at[si[0]], rows[0], gsem[0])
      pltpu.make_async_copy(feat.at[si[1]], rows[1], gsem[1]).wait()
      pltpu.async_copy(rows[1], acc.at[di[1]], ssem[1], add=True)
      return carry

    lax.fori_loop(0, nch // 2, outer, 0)
    pltpu.make_async_copy(rows[0], acc.at[di[0]], ssem[0]).wait()
    pltpu.make_async_copy(rows[1], acc.at[di[1]], ssem[1]).wait()
    plsc.subcore_barrier()

    obase = cid * NP + rbase
    for t in range(RPT // CHUNK):
      pltpu.sync_copy(acc.at[pl.ds(rbase + t * CHUNK, CHUNK)], r0)
      pltpu.sync_copy(r0, sums_out.at[pl.ds(obase + t * CHUNK, CHUNK)])
    tail = RPT - (RPT // CHUNK) * CHUNK
    toff = (RPT // CHUNK) * CHUNK
    pltpu.sync_copy(acc.at[pl.ds(rbase + toff, tail)], r0.at[pl.ds(0, tail)])
    pltpu.sync_copy(r0.at[pl.ds(0, tail)],
                    sums_out.at[pl.ds(obase + toff, tail)])

  return pl.kernel(
      body,
      out_type=jax.ShapeDtypeStruct((NC * NP, D), jnp.float32),
      mesh=_MESH,
      scratch_types=[
          pltpu.VMEM((2, CHUNK), jnp.int32),
          pltpu.VMEM((2, CHUNK), jnp.int32),
          pltpu.VMEM((CHUNK, D), jnp.float32),
          pltpu.VMEM((CHUNK, D), jnp.float32),
          pltpu.VMEM_SHARED((NP, D), jnp.float32),
          pltpu.SemaphoreType.DMA,
          pltpu.SemaphoreType.DMA,
          pltpu.SemaphoreType.DMA,
          pltpu.SemaphoreType.DMA,
      ],
      compiler_params=_SC_PARAMS,
  )


RB = 1024


def _make_combine(relu: bool):
  def body(s_ref, c_ref, x_ref, wl_ref, b_ref, wr_ref, o_ref):
    s = s_ref[0] + s_ref[1]
    c = c_ref[0] + c_ref[1]
    mean = s / jnp.maximum(c, 1.0)
    acc = lax.dot_general(mean, wl_ref[...], (((1,), (1,)), ((), ())),
                          preferred_element_type=jnp.float32)
    acc = acc + lax.dot_general(x_ref[...], wr_ref[...],
                                (((1,), (1,)), ((), ())),
                                preferred_element_type=jnp.float32)
    acc = acc + b_ref[...]
    o_ref[...] = jnp.maximum(acc, 0.0) if relu else acc

  return pl.pallas_call(
      body,
      grid=(NP // RB,),
      in_specs=[
          pl.BlockSpec((NC, RB, D), lambda i: (0, i, 0)),
          pl.BlockSpec((NC, RB, 1), lambda i: (0, i, 0)),
          pl.BlockSpec((RB, D), lambda i: (i, 0)),
          pl.BlockSpec((D, D), lambda i: (0, 0)),
          pl.BlockSpec((1, D), lambda i: (0, 0)),
          pl.BlockSpec((D, D), lambda i: (0, 0)),
      ],
      out_specs=pl.BlockSpec((RB, D), lambda i: (i, 0)),
      out_shape=jax.ShapeDtypeStruct((NP, D), jnp.float32),
  )


_degree = _make_degree()
_agg = _make_agg()
_combine_relu = _make_combine(True)
_combine_lin = _make_combine(False)


def kernel(x, edge_index, W_l1, b_l1, W_r1, W_l2, b_l2, W_r2):
  src = edge_index[0].astype(jnp.int32)
  dst = edge_index[1].astype(jnp.int32)
  pad = EPAD - N_EDGES
  src_p = jnp.concatenate([src, jnp.zeros((pad,), jnp.int32)])
  dst_p = jnp.concatenate([dst, jnp.full((pad,), NP - 8, jnp.int32)])
  x_p = jnp.pad(x, ((0, NP - N_NODES), (0, 0)))

  cnt = _degree(dst_p)
  c1 = cnt.reshape(NC, NP, 1)

  il = jnp.stack([src_p.reshape(TOTCH, CHUNK), dst_p.reshape(TOTCH, CHUNK)],
                 axis=1)
  sums1 = _agg(x_p, il)
  h = _combine_relu(sums1.reshape(NC, NP, D), c1, x_p,
                    W_l1, b_l1.reshape(1, D), W_r1)

  sums2 = _agg(h, il)
  out = _combine_lin(sums2.reshape(NC, NP, D), c1, h,
                     W_l2, b_l2.reshape(1, D), W_r2)
  return out[:N_NODES]
